# Initial kernel scaffold; baseline (speedup 1.0000x reference)
import jax
import jax.numpy as jnp
from jax.experimental import pallas as pl


def kernel(n_input_all, t_input_all, embN, embT, Wih, Whh, bih, bhh, Wih_d, Whh_d, bih_d, bhh_d, hid_init, cell_init, dyn_init_h, dyn_init_c):
    raise NotImplementedError("write your pallas kernel here")


# speedup vs baseline: 9.1224x; 9.1224x over previous
import functools

import jax
import jax.numpy as jnp
from jax import lax
from jax.experimental import pallas as pl
from jax.experimental.pallas import tpu as pltpu
from jax.experimental.pallas import tpu_sc as plsc

B, T = 128, 50
N_STATIC = 64
EN, ET, H = 64, 128, 512
ENP = 128
D = ET
G = B * T

_NC, _NS = 2, 16
_NW = _NC * _NS
_PW = G // _NW
_CHUNKS = ((0, 128), (128, 72))


def _sc_gather_body(embN_hbm, nidx_hbm, embT_hbm, tidx_hbm,
                    outN_hbm, outT_hbm,
                    idxN_v, rowsN_v, idxT_v, rowsT_v, semN, semT):
    wid = lax.axis_index("s") * _NC + lax.axis_index("c")
    base = wid * _PW
    pltpu.sync_copy(nidx_hbm.at[pl.ds(base, _PW)], idxN_v)
    pltpu.sync_copy(tidx_hbm.at[pl.ds(base, _PW)], idxT_v)
    copies = []
    for off, width in _CHUNKS:
        copies.append(pltpu.async_copy(
            embN_hbm.at[idxN_v.at[pl.ds(off, width)]],
            rowsN_v.at[pl.ds(off, width)], semN))
        copies.append(pltpu.async_copy(
            embT_hbm.at[idxT_v.at[pl.ds(off, width)]],
            rowsT_v.at[pl.ds(off, width)], semT))
    for cp in copies:
        cp.wait()
    pltpu.sync_copy(rowsN_v, outN_hbm.at[pl.ds(base, _PW)])
    pltpu.sync_copy(rowsT_v, outT_hbm.at[pl.ds(base, _PW)])


def _sc_gather(embN, nidx, embT, tidx):
    gather = functools.partial(
        pl.kernel,
        out_type=(jax.ShapeDtypeStruct((G, ENP), jnp.float32),
                  jax.ShapeDtypeStruct((G, ET), jnp.float32)),
        mesh=plsc.VectorSubcoreMesh(core_axis_name="c", subcore_axis_name="s",
                                    num_cores=_NC),
        scratch_types=[
            pltpu.VMEM((_PW,), jnp.int32),
            pltpu.VMEM((_PW, ENP), jnp.float32),
            pltpu.VMEM((_PW,), jnp.int32),
            pltpu.VMEM((_PW, ET), jnp.float32),
            pltpu.SemaphoreType.DMA,
            pltpu.SemaphoreType.DMA,
        ],
    )(_sc_gather_body)
    return gather(embN, nidx, embT, tidx)


def _tc_body(tfull_ref, nemb_ref, semb_ref,
             Wn_ref, Wht_ref, Wh_ref, bm_ref,
             Wdn_ref, Wdh_ref, Wdhd_ref, bd_ref,
             h0_ref, c0_ref, dih_ref, dic_ref,
             out_ref,
             h_s, c_s, hist_s, prev_s, need_s, red_s):
    ts = pl.program_id(0)
    lane = lax.broadcasted_iota(jnp.int32, (B, T), 1)

    @pl.when(ts == 0)
    def _prologue():
        h_s[...] = jnp.broadcast_to(h0_ref[...], (B, H))
        c_s[...] = jnp.broadcast_to(c0_ref[...], (B, H))
        t_all = tfull_ref[...]
        d = jnp.where(t_all >= N_STATIC, t_all, -1)
        prev = jnp.full((B, T), -1, jnp.int32)
        for tp in range(T - 1):
            dcol = d[:, tp:tp + 1]
            hit = (d == dcol) & (lane > tp) & (dcol >= 0)
            prev = jnp.where(hit, tp, prev)
        prev_s[...] = prev
        need = jnp.zeros((B, T), jnp.int32)
        for t2 in range(1, T):
            need = jnp.where(lane == prev[:, t2:t2 + 1], 1, need)
        need_s[...] = need

    sel = lane == ts
    prev_col = jnp.sum(jnp.where(sel, prev_s[...], 0), axis=1, keepdims=True)
    upd_col = prev_col >= 0
    r_any = jnp.sum(upd_col.astype(jnp.int32)) > 0
    w_any = jnp.sum(jnp.where(sel, need_s[...], 0)) > 0
    t_col = jnp.sum(jnp.where(sel, tfull_ref[...], 0), axis=1, keepdims=True)

    @pl.when(r_any)
    def _reduce():
        iota3 = lax.broadcasted_iota(jnp.int32, (T, B, 1), 0)
        eq3 = iota3 == prev_col[None, :, :]
        red_s[...] = jnp.sum(jnp.where(eq3, hist_s[...], 0.0), axis=0)

    n_emb = nemb_ref[...].reshape(B, ENP)
    s_emb = semb_ref[...].reshape(B, ET)
    red = red_s[...]
    init_h = jnp.broadcast_to(dih_ref[...], (B, D))
    init_c = jnp.broadcast_to(dic_ref[...], (B, D))
    h_dynamic = jnp.where(upd_col, red[:, :D], init_h)
    c_dynamic = jnp.where(upd_col, red[:, D:], init_c)
    h_tensor = jnp.where(t_col < N_STATIC, s_emb, h_dynamic)

    h = h_s[...]
    c = c_s[...]
    g = (jnp.dot(n_emb, Wn_ref[...], preferred_element_type=jnp.float32)
         + jnp.dot(h_tensor, Wht_ref[...], preferred_element_type=jnp.float32)
         + jnp.dot(h, Wh_ref[...], preferred_element_type=jnp.float32)
         + bm_ref[...])
    i_g = jax.nn.sigmoid(g[:, 0 * H:1 * H])
    f_g = jax.nn.sigmoid(g[:, 1 * H:2 * H])
    g_g = jnp.tanh(g[:, 2 * H:3 * H])
    o_g = jax.nn.sigmoid(g[:, 3 * H:4 * H])
    c_new = f_g * c + i_g * g_g
    h_new = o_g * jnp.tanh(c_new)
    out_ref[...] = h_new[None]
    h_s[...] = h_new
    c_s[...] = c_new

    @pl.when(w_any)
    def _dyn():
        gd = (jnp.dot(n_emb, Wdn_ref[...], preferred_element_type=jnp.float32)
              + jnp.dot(h, Wdh_ref[...], preferred_element_type=jnp.float32)
              + jnp.dot(h_dynamic, Wdhd_ref[...],
                        preferred_element_type=jnp.float32)
              + bd_ref[...])
        i_d = jax.nn.sigmoid(gd[:, 0 * D:1 * D])
        f_d = jax.nn.sigmoid(gd[:, 1 * D:2 * D])
        g_d = jnp.tanh(gd[:, 2 * D:3 * D])
        o_d = jax.nn.sigmoid(gd[:, 3 * D:4 * D])
        c_dyn = f_d * c_dynamic + i_d * g_d
        h_dyn = o_d * jnp.tanh(c_dyn)
        hist_s[pl.ds(ts, 1)] = jnp.concatenate([h_dyn, c_dyn], axis=1)[None]


def _tc_forward(tfull, nemb3, semb3, Wn, Wht, Wh, bm, Wdn, Wdh, Wdhd, bd,
                h0, c0, dih, dic):
    full = lambda shape: pl.BlockSpec(shape, lambda t: (0,) * len(shape))
    return pl.pallas_call(
        _tc_body,
        grid=(T,),
        in_specs=[
            full((B, T)),
            pl.BlockSpec((1, B, ENP), lambda t: (t, 0, 0)),
            pl.BlockSpec((1, B, ET), lambda t: (t, 0, 0)),
            full((ENP, 4 * H)),
            full((ET, 4 * H)),
            full((H, 4 * H)),
            full((1, 4 * H)),
            full((ENP, 4 * D)),
            full((H, 4 * D)),
            full((D, 4 * D)),
            full((1, 4 * D)),
            full((1, H)),
            full((1, H)),
            full((1, D)),
            full((1, D)),
        ],
        out_specs=pl.BlockSpec((1, B, H), lambda t: (t, 0, 0)),
        out_shape=jax.ShapeDtypeStruct((T, B, H), jnp.float32),
        scratch_shapes=[
            pltpu.VMEM((B, H), jnp.float32),
            pltpu.VMEM((B, H), jnp.float32),
            pltpu.VMEM((T, B, 2 * D), jnp.float32),
            pltpu.VMEM((B, T), jnp.int32),
            pltpu.VMEM((B, T), jnp.int32),
            pltpu.VMEM((B, 2 * D), jnp.float32),
        ],
        compiler_params=pltpu.CompilerParams(
            dimension_semantics=("arbitrary",)),
    )(tfull, nemb3, semb3, Wn, Wht, Wh, bm, Wdn, Wdh, Wdhd, bd,
      h0, c0, dih, dic)


def kernel(n_input_all, t_input_all, embN, embT, Wih, Whh, bih, bhh,
           Wih_d, Whh_d, bih_d, bhh_d, hid_init, cell_init,
           dyn_init_h, dyn_init_c):
    nidx = n_input_all.T.reshape(G).astype(jnp.int32)
    tidx = jnp.clip(t_input_all.T, 0, N_STATIC - 1).reshape(G).astype(jnp.int32)
    embN_p = jnp.concatenate(
        [embN, jnp.zeros((embN.shape[0], ENP - EN), jnp.float32)], axis=1)
    nemb_rows, semb_rows = _sc_gather(embN_p, nidx, embT, tidx)
    nemb3 = nemb_rows.reshape(T, B, ENP)
    semb3 = semb_rows.reshape(T, B, ET)

    tfull = t_input_all.astype(jnp.int32)
    Wn = jnp.concatenate(
        [Wih[:, :EN].T, jnp.zeros((ENP - EN, 4 * H), jnp.float32)], axis=0)
    Wht = Wih[:, EN:].T
    Wh = Whh.T
    bm = (bih + bhh)[None, :]
    Wdn = jnp.concatenate(
        [Wih_d[:, :EN].T, jnp.zeros((ENP - EN, 4 * D), jnp.float32)], axis=0)
    Wdh = Wih_d[:, EN:].T
    Wdhd = Whh_d.T
    bd = (bih_d + bhh_d)[None, :]
    h0 = hid_init[None, :]
    c0 = cell_init[None, :]

    out_t = _tc_forward(tfull, nemb3, semb3, Wn, Wht, Wh, bm,
                        Wdn, Wdh, Wdhd, bd, h0, c0, dyn_init_h, dyn_init_c)
    return jnp.transpose(out_t, (1, 0, 2))
